# baseline (speedup 1.0000x reference)
import functools

import jax
import jax.numpy as jnp
from jax.experimental import pallas as pl

B, D, Z, H, K = 4096, 10000, 32, 128, 16
BLK = 128


def _body(x_ref, w1_ref, b1_ref, wmu_ref, bmu_ref, emb_ref,
          de1_ref, deb1_ref, de2_ref, deb2_ref,
          dq1_ref, dqb1_ref, dq2_ref, dqb2_ref,
          xe_ref, xq_ref, ze_ref, zq_ref, k_ref, zd_ref, dp_ref):
    f32 = jnp.float32
    x = x_ref[...]
    h = jnp.maximum(
        jnp.dot(x, w1_ref[...], preferred_element_type=f32) + b1_ref[...], 0.0)
    z_e = jnp.dot(h, wmu_ref[...], preferred_element_type=f32) + bmu_ref[...]
    emb = emb_ref[...]

    cols = []
    for j in range(K):
        dj = jnp.sum((z_e - emb[j:j + 1, :]) ** 2, axis=1, keepdims=True)
        cols.append(dj)
    z_dist = jnp.concatenate(cols, axis=1)

    prob = jnp.power(1.0 + z_dist / 10.0, -5.5)
    dist_prob = prob / jnp.sum(prob, axis=1, keepdims=True)

    idx16 = jax.lax.broadcasted_iota(jnp.int32, (BLK, K), 1)
    mx = jnp.max(dist_prob, axis=1, keepdims=True)
    kk = jnp.min(jnp.where(dist_prob == mx, idx16, K), axis=1, keepdims=True)
    onehot = (idx16 == kk).astype(f32)

    z_q = jnp.dot(onehot, emb, preferred_element_type=f32)

    he = jnp.maximum(
        jnp.dot(z_e, de1_ref[...], preferred_element_type=f32) + deb1_ref[...], 0.0)
    x_e = jnp.dot(he, de2_ref[...], preferred_element_type=f32) + deb2_ref[...]

    cb_h = jnp.maximum(
        jnp.dot(emb, dq1_ref[...], preferred_element_type=f32) + dqb1_ref[...], 0.0)
    codebook = jnp.dot(cb_h, dq2_ref[...], preferred_element_type=f32) + dqb2_ref[...]
    x_q = jnp.dot(onehot, codebook, preferred_element_type=f32)

    xe_ref[...] = x_e
    xq_ref[...] = x_q
    ze_ref[...] = z_e
    zq_ref[...] = z_q
    k_ref[...] = kk
    zd_ref[...] = z_dist
    dp_ref[...] = dist_prob


@jax.jit
def _run(x, enc_W1, enc_b1, enc_Wmu, enc_bmu, embeddings,
         dece_W1, dece_b1, dece_W2, dece_b2,
         decq_W1, decq_b1, decq_W2, decq_b2):
    nb = B // BLK
    full = lambda shape: pl.BlockSpec(shape, lambda i: (0,) * len(shape))
    row = lambda w: pl.BlockSpec((BLK, w), lambda i: (i, 0))
    out_shapes = (
        jax.ShapeDtypeStruct((B, D), jnp.float32),
        jax.ShapeDtypeStruct((B, D), jnp.float32),
        jax.ShapeDtypeStruct((B, Z), jnp.float32),
        jax.ShapeDtypeStruct((B, Z), jnp.float32),
        jax.ShapeDtypeStruct((B, 1), jnp.int32),
        jax.ShapeDtypeStruct((B, K), jnp.float32),
        jax.ShapeDtypeStruct((B, K), jnp.float32),
    )
    return pl.pallas_call(
        _body,
        grid=(nb,),
        in_specs=[
            row(D),
            full((D, H)), full((1, H)), full((H, Z)), full((1, Z)),
            full((K, Z)),
            full((Z, H)), full((1, H)), full((H, D)), full((1, D)),
            full((Z, H)), full((1, H)), full((H, D)), full((1, D)),
        ],
        out_specs=[
            row(D), row(D), row(Z), row(Z), row(1), row(K), row(K),
        ],
        out_shape=out_shapes,
    )(x, enc_W1, enc_b1, enc_Wmu, enc_bmu, embeddings,
      dece_W1, dece_b1, dece_W2, dece_b2,
      decq_W1, decq_b1, decq_W2, decq_b2)


def kernel(x, enc_W1, enc_b1, enc_Wmu, enc_bmu, embeddings,
           dece_W1, dece_b1, dece_W2, dece_b2,
           decq_W1, decq_b1, decq_W2, decq_b2):
    x_e, x_q, z_e, z_q, k2, z_dist, dist_prob = _run(
        x, enc_W1, enc_b1.reshape(1, H), enc_Wmu, enc_bmu.reshape(1, Z),
        embeddings,
        dece_W1, dece_b1.reshape(1, H), dece_W2, dece_b2.reshape(1, D),
        decq_W1, decq_b1.reshape(1, H), decq_W2, decq_b2.reshape(1, D))
    return (x_e, x_q, z_e, z_q, k2[:, 0], z_dist, dist_prob)
